# Initial kernel scaffold; baseline (speedup 1.0000x reference)
import jax
import jax.numpy as jnp
from jax.experimental import pallas as pl


def kernel(x, batch, weight, bias):
    raise NotImplementedError("write your pallas kernel here")


# speedup vs baseline: 32.5327x; 32.5327x over previous
import jax
import jax.numpy as jnp
from jax.experimental import pallas as pl

_NSEG = 512
_EPS = 1e-05
_R = 512


def _phase1_body(batch_ref, x_ref, s1_ref, sq_ref):
    i = pl.program_id(0)
    ids = batch_ref[0]
    oh = (jax.lax.broadcasted_iota(jnp.int32, (_NSEG, _R), 0) == ids).astype(
        jnp.float32)
    xs = x_ref[...]
    x64 = xs[:, :64]
    onecol = (jax.lax.broadcasted_iota(jnp.int32, (_R, 64), 1) == 0).astype(
        jnp.float32)
    payload = jnp.concatenate([x64 * x64, onecol], axis=1)

    @pl.when(i == 0)
    def _():
        s1_ref[...] = jnp.zeros_like(s1_ref)
        sq_ref[...] = jnp.zeros_like(sq_ref)

    s1_ref[...] += jax.lax.dot(oh, xs, preferred_element_type=jnp.float32)
    sq_ref[...] += jax.lax.dot(oh, payload, preferred_element_type=jnp.float32)


def _tables_body(s1_ref, sq_ref, wcol_ref, bcol_ref, a_ref, b_ref):
    s1 = s1_ref[...]
    sq = sq_ref[...]
    deg = sq[:, 64:65]
    dc = jnp.maximum(deg, 1.0)
    mean = s1 / (deg + 1e-12)
    m64 = mean[:, :64]
    var = (sq[:, :64] - 2.0 * m64 * s1[:, :64] + deg * m64 * m64) / dc
    norm = jnp.sum(var, axis=1, keepdims=True) * (1.0 / 64.0)
    inv = 1.0 / (jnp.sqrt(norm) + _EPS)
    colmask = jax.lax.broadcasted_iota(jnp.int32, (_NSEG, 240), 1) < 64
    s_full = jnp.where(colmask, inv, 1.0)
    a = s_full * wcol_ref[...]
    a_ref[...] = a
    b_ref[...] = bcol_ref[...] - mean * a


def _phase2_body(batch_ref, x_ref, a_ref, b_ref, o_ref):
    ids = batch_ref[0]
    oh = (jax.lax.broadcasted_iota(jnp.int32, (_NSEG, _R), 0) == ids).astype(
        jnp.float32)
    dn = (((0,), (0,)), ((), ()))
    ga = jax.lax.dot_general(oh, a_ref[...], dn,
                             preferred_element_type=jnp.float32)
    gb = jax.lax.dot_general(oh, b_ref[...], dn,
                             preferred_element_type=jnp.float32)
    o_ref[...] = x_ref[...] * ga + gb


def kernel(x, batch, weight, bias):
    n, tot = x.shape
    assert n % _R == 0
    nblk = n // _R
    batch3 = batch.reshape(nblk, 1, _R)

    wcol = jnp.concatenate([
        weight[0:64],
        jnp.repeat(weight[64:96], 3),
        jnp.repeat(weight[96:112], 5),
    ]).reshape(1, tot)
    bcol = jnp.concatenate([bias, jnp.zeros((tot - 64,), jnp.float32)
                            ]).reshape(1, tot)

    s1, sq = pl.pallas_call(
        _phase1_body,
        grid=(nblk,),
        in_specs=[
            pl.BlockSpec((1, 1, _R), lambda i: (i, 0, 0)),
            pl.BlockSpec((_R, tot), lambda i: (i, 0)),
        ],
        out_specs=[
            pl.BlockSpec((_NSEG, tot), lambda i: (0, 0)),
            pl.BlockSpec((_NSEG, 128), lambda i: (0, 0)),
        ],
        out_shape=[
            jax.ShapeDtypeStruct((_NSEG, tot), jnp.float32),
            jax.ShapeDtypeStruct((_NSEG, 128), jnp.float32),
        ],
    )(batch3, x)

    a_tab, b_tab = pl.pallas_call(
        _tables_body,
        out_shape=[
            jax.ShapeDtypeStruct((_NSEG, tot), jnp.float32),
            jax.ShapeDtypeStruct((_NSEG, tot), jnp.float32),
        ],
    )(s1, sq, wcol, bcol)

    out = pl.pallas_call(
        _phase2_body,
        grid=(nblk,),
        in_specs=[
            pl.BlockSpec((1, 1, _R), lambda i: (i, 0, 0)),
            pl.BlockSpec((_R, tot), lambda i: (i, 0)),
            pl.BlockSpec((_NSEG, tot), lambda i: (0, 0)),
            pl.BlockSpec((_NSEG, tot), lambda i: (0, 0)),
        ],
        out_specs=pl.BlockSpec((_R, tot), lambda i: (i, 0)),
        out_shape=jax.ShapeDtypeStruct((n, tot), jnp.float32),
    )(batch3, x, a_tab, b_tab)
    return out
